# baseline (speedup 1.0000x reference)
import jax
import jax.numpy as jnp
from jax.experimental import pallas as pl
from jax.experimental.pallas import tpu as pltpu

_B = 16
_A = 3
_C = 21
_LEVELS = ((64, 64), (32, 32), (16, 16), (8, 8))
_NBOX = 20
_IOU_THR = 0.5


def _match_shape(H, W):
    return H, W


def _level_losses(anch_ref, loc_ref, conf_ref, bxs, H, W):
    loc_sum = 0.0
    nm = 0.0
    ce = 0.0
    packed = _match_shape(H, W) != (H, W)
    for a in range(_A):
        ax1 = anch_ref[4 * a + 0]
        ay1 = anch_ref[4 * a + 1]
        ax2 = anch_ref[4 * a + 2]
        ay2 = anch_ref[4 * a + 3]
        area_a = (ax2 - ax1) * (ay2 - ay1)

        best = jnp.full(ax1.shape, -jnp.inf, jnp.float32)
        zero = jnp.zeros(ax1.shape, jnp.float32)
        mx1, my1, mx2, my2 = zero, zero, zero, zero
        for j in range(_NBOX):
            bx1, by1, bx2, by2, barea = bxs[j]
            iw = jnp.maximum(jnp.minimum(ax2, bx2) - jnp.maximum(ax1, bx1), 0.0)
            ih = jnp.maximum(jnp.minimum(ay2, by2) - jnp.maximum(ay1, by1), 0.0)
            inter = iw * ih
            union = (area_a + barea) - inter
            iou = inter / (union + 1e-6)
            upd = iou > best
            best = jnp.where(upd, iou, best)
            mx1 = jnp.where(upd, bx1, mx1)
            my1 = jnp.where(upd, by1, my1)
            mx2 = jnp.where(upd, bx2, mx2)
            my2 = jnp.where(upd, by2, my2)

        pos = best >= _IOU_THR
        posf = pos.astype(jnp.float32)
        aw = ax2 - ax1
        ah = ay2 - ay1
        safe_aw = jnp.where(pos, aw, 1.0)
        safe_ah = jnp.where(pos, ah, 1.0)
        ocx = ((mx1 + mx2) * 0.5 - (ax1 + ax2) * 0.5) / safe_aw
        ocy = ((my1 + my2) * 0.5 - (ay1 + ay2) * 0.5) / safe_ah
        rw = jnp.where(pos, (mx2 - mx1) / safe_aw, 1.0)
        rh = jnp.where(pos, (my2 - my1) / safe_ah, 1.0)
        rw = jnp.where(rw > 0.0, rw, 1.0)
        rh = jnp.where(rh > 0.0, rh, 1.0)
        ow = jnp.log(rw)
        oh = jnp.log(rh)

        if packed:
            posf = posf.reshape(H, W)
            ocx = ocx.reshape(H, W)
            ocy = ocy.reshape(H, W)
            ow = ow.reshape(H, W)
            oh = oh.reshape(H, W)
            pos = posf > 0.0

        def sl1(d):
            ad = jnp.abs(d)
            return jnp.where(ad < 1.0, 0.5 * d * d, ad - 0.5)

        sl = sl1(loc_ref[0, 4 * a + 0] - ocx) \
            + sl1(loc_ref[0, 4 * a + 1] - ocy) \
            + sl1(loc_ref[0, 4 * a + 2] - ow) \
            + sl1(loc_ref[0, 4 * a + 3] - oh)
        loc_sum += jnp.sum(sl * posf)
        nm += jnp.sum(posf)

        p0 = conf_ref[0, _C * a]
        p1 = conf_ref[0, _C * a + 1]
        m = jnp.maximum(p0, p1)
        planes = [conf_ref[0, _C * a + c] for c in range(2, _C)]
        for p in planes:
            m = jnp.maximum(m, p)
        s = jnp.exp(p0 - m) + jnp.exp(p1 - m)
        for p in planes:
            s += jnp.exp(p - m)
        lse = m + jnp.log(s)
        picked = jnp.where(pos, p1, p0)
        ce += jnp.sum(lse - picked)
    return loc_sum, nm, ce


def _fused_kernel(box_ref,
                  a0, l0, c0, a1, l1, c1, a2, l2, c2, a3, l3, c3,
                  loc_out, nm_out, ce_out):
    b = pl.program_id(0)

    @pl.when(b == 0)
    def _init():
        loc_out[0, 0] = 0.0
        nm_out[0, 0] = 0.0
        ce_out[0, 0] = 0.0

    bxs = []
    for j in range(_NBOX):
        bx1 = box_ref[0, j, 0]
        by1 = box_ref[0, j, 1]
        bx2 = box_ref[0, j, 2]
        by2 = box_ref[0, j, 3]
        bxs.append((bx1, by1, bx2, by2, (bx2 - bx1) * (by2 - by1)))

    loc_t = 0.0
    nm_t = 0.0
    ce_t = 0.0
    for (H, W), anch, loc, conf in zip(
            _LEVELS,
            (a0, a1, a2, a3), (l0, l1, l2, l3), (c0, c1, c2, c3)):
        ls, nm, ce = _level_losses(anch, loc, conf, bxs, H, W)
        loc_t += ls
        nm_t += nm
        ce_t += ce

    loc_out[0, 0] += loc_t
    nm_out[0, 0] += nm_t
    ce_out[0, 0] += ce_t


def kernel(odm_loc_0, odm_loc_1, odm_loc_2, odm_loc_3,
           odm_conf_0, odm_conf_1, odm_conf_2, odm_conf_3,
           gt_boxes, gt_labels, anchors):
    del gt_labels

    anch_levels = []
    start = 0
    for H, W in _LEVELS:
        hb, lw = _match_shape(H, W)
        n = H * W * _A
        anch_levels.append(
            anchors[start:start + n]
            .reshape(H * W, _A, 4)
            .transpose(1, 2, 0)
            .reshape(_A * 4, hb, lw))
        start += n

    locs = (odm_loc_0, odm_loc_1, odm_loc_2, odm_loc_3)
    confs = (odm_conf_0, odm_conf_1, odm_conf_2, odm_conf_3)

    in_specs = [pl.BlockSpec((1, _NBOX, 4), lambda b: (b, 0, 0),
                             memory_space=pltpu.SMEM)]
    operands = [gt_boxes]
    for i, (H, W) in enumerate(_LEVELS):
        hb, lw = _match_shape(H, W)
        in_specs.append(pl.BlockSpec((_A * 4, hb, lw), lambda b: (0, 0, 0)))
        operands.append(anch_levels[i])
        in_specs.append(
            pl.BlockSpec((1, _A * 4, H, W), lambda b: (b, 0, 0, 0)))
        operands.append(locs[i])
        in_specs.append(
            pl.BlockSpec((1, _A * _C, H, W), lambda b: (b, 0, 0, 0)))
        operands.append(confs[i])

    scalar_spec = pl.BlockSpec((1, 1), lambda b: (0, 0),
                               memory_space=pltpu.SMEM)
    out = pl.pallas_call(
        _fused_kernel,
        grid=(_B,),
        in_specs=in_specs,
        out_specs=[scalar_spec, scalar_spec, scalar_spec],
        out_shape=[jax.ShapeDtypeStruct((1, 1), jnp.float32)] * 3,
    )(*operands)

    loc_s = out[0][0, 0]
    nm = out[1][0, 0]
    ce = out[2][0, 0]
    total_loc = jnp.where(nm > 0.0, loc_s / jnp.maximum(nm, 1.0), loc_s)
    return total_loc + ce / float(_B)


# speedup vs baseline: 1.1893x; 1.1893x over previous
import jax
import jax.numpy as jnp
from jax.experimental import pallas as pl
from jax.experimental.pallas import tpu as pltpu

_B = 16
_A = 3
_C = 21
_LEVELS = ((64, 64), (32, 32), (16, 16), (8, 8))
_NBOX = 20
_IOU_THR = 0.5


def _plane_shape(H, W):
    hw = H * W
    if hw >= 128:
        return hw // 128, 128
    return 1, hw


def _level_losses(anch_ref, loc_ref, conf_ref, bxs):
    loc_sum = 0.0
    nm = 0.0
    ce = 0.0
    for a in range(_A):
        ax1 = anch_ref[4 * a + 0]
        ay1 = anch_ref[4 * a + 1]
        ax2 = anch_ref[4 * a + 2]
        ay2 = anch_ref[4 * a + 3]
        area_a = (ax2 - ax1) * (ay2 - ay1)

        best = jnp.full(ax1.shape, -jnp.inf, jnp.float32)
        zero = jnp.zeros(ax1.shape, jnp.float32)
        mx1, my1, mx2, my2 = zero, zero, zero, zero
        for j in range(_NBOX):
            bx1, by1, bx2, by2, barea = bxs[j]
            iw = jnp.maximum(jnp.minimum(ax2, bx2) - jnp.maximum(ax1, bx1), 0.0)
            ih = jnp.maximum(jnp.minimum(ay2, by2) - jnp.maximum(ay1, by1), 0.0)
            inter = iw * ih
            union = (area_a + barea) - inter
            iou = inter / (union + 1e-6)
            upd = iou > best
            best = jnp.where(upd, iou, best)
            mx1 = jnp.where(upd, bx1, mx1)
            my1 = jnp.where(upd, by1, my1)
            mx2 = jnp.where(upd, bx2, mx2)
            my2 = jnp.where(upd, by2, my2)

        pos = best >= _IOU_THR
        posf = pos.astype(jnp.float32)
        aw = ax2 - ax1
        ah = ay2 - ay1
        safe_aw = jnp.where(pos, aw, 1.0)
        safe_ah = jnp.where(pos, ah, 1.0)
        ocx = ((mx1 + mx2) * 0.5 - (ax1 + ax2) * 0.5) / safe_aw
        ocy = ((my1 + my2) * 0.5 - (ay1 + ay2) * 0.5) / safe_ah
        rw = jnp.where(pos, (mx2 - mx1) / safe_aw, 1.0)
        rh = jnp.where(pos, (my2 - my1) / safe_ah, 1.0)
        rw = jnp.where(rw > 0.0, rw, 1.0)
        rh = jnp.where(rh > 0.0, rh, 1.0)
        ow = jnp.log(rw)
        oh = jnp.log(rh)

        def sl1(d):
            ad = jnp.abs(d)
            return jnp.where(ad < 1.0, 0.5 * d * d, ad - 0.5)

        sl = sl1(loc_ref[0, 4 * a + 0] - ocx) \
            + sl1(loc_ref[0, 4 * a + 1] - ocy) \
            + sl1(loc_ref[0, 4 * a + 2] - ow) \
            + sl1(loc_ref[0, 4 * a + 3] - oh)
        loc_sum += jnp.sum(sl * posf)
        nm += jnp.sum(posf)

        p0 = conf_ref[0, _C * a]
        p1 = conf_ref[0, _C * a + 1]
        m = jnp.maximum(p0, p1)
        planes = [conf_ref[0, _C * a + c] for c in range(2, _C)]
        for p in planes:
            m = jnp.maximum(m, p)
        s = jnp.exp(p0 - m) + jnp.exp(p1 - m)
        for p in planes:
            s += jnp.exp(p - m)
        lse = m + jnp.log(s)
        picked = jnp.where(pos, p1, p0)
        ce += jnp.sum(lse - picked)
    return loc_sum, nm, ce


def _fused_kernel(box_ref,
                  a0, l0, c0, a1, l1, c1, a2, l2, c2, a3, l3, c3,
                  loc_out, nm_out, ce_out):
    b = pl.program_id(0)

    @pl.when(b == 0)
    def _init():
        loc_out[0, 0] = 0.0
        nm_out[0, 0] = 0.0
        ce_out[0, 0] = 0.0

    bxs = []
    for j in range(_NBOX):
        bx1 = box_ref[0, j, 0]
        by1 = box_ref[0, j, 1]
        bx2 = box_ref[0, j, 2]
        by2 = box_ref[0, j, 3]
        bxs.append((bx1, by1, bx2, by2, (bx2 - bx1) * (by2 - by1)))

    loc_t = 0.0
    nm_t = 0.0
    ce_t = 0.0
    for anch, loc, conf in ((a0, l0, c0), (a1, l1, c1),
                            (a2, l2, c2), (a3, l3, c3)):
        ls, nm, ce = _level_losses(anch, loc, conf, bxs)
        loc_t += ls
        nm_t += nm
        ce_t += ce

    loc_out[0, 0] += loc_t
    nm_out[0, 0] += nm_t
    ce_out[0, 0] += ce_t


def kernel(odm_loc_0, odm_loc_1, odm_loc_2, odm_loc_3,
           odm_conf_0, odm_conf_1, odm_conf_2, odm_conf_3,
           gt_boxes, gt_labels, anchors):
    del gt_labels

    at4 = anchors.T
    anch_levels = []
    start = 0
    for H, W in _LEVELS:
        hb, lw = _plane_shape(H, W)
        n = H * W * _A
        groups = [at4[:, start + a:start + n:_A].reshape(4, hb, lw)
                  for a in range(_A)]
        anch_levels.append(jnp.stack(groups).reshape(_A * 4, hb, lw))
        start += n

    locs = (odm_loc_0, odm_loc_1, odm_loc_2, odm_loc_3)
    confs = (odm_conf_0, odm_conf_1, odm_conf_2, odm_conf_3)

    in_specs = [pl.BlockSpec((1, _NBOX, 4), lambda b: (b, 0, 0),
                             memory_space=pltpu.SMEM)]
    operands = [gt_boxes]
    for i, (H, W) in enumerate(_LEVELS):
        hb, lw = _plane_shape(H, W)
        in_specs.append(pl.BlockSpec((_A * 4, hb, lw), lambda b: (0, 0, 0)))
        operands.append(anch_levels[i])
        in_specs.append(
            pl.BlockSpec((1, _A * 4, hb, lw), lambda b: (b, 0, 0, 0)))
        operands.append(locs[i].reshape(_B, _A * 4, hb, lw))
        in_specs.append(
            pl.BlockSpec((1, _A * _C, hb, lw), lambda b: (b, 0, 0, 0)))
        operands.append(confs[i].reshape(_B, _A * _C, hb, lw))

    scalar_spec = pl.BlockSpec((1, 1), lambda b: (0, 0),
                               memory_space=pltpu.SMEM)
    out = pl.pallas_call(
        _fused_kernel,
        grid=(_B,),
        in_specs=in_specs,
        out_specs=[scalar_spec, scalar_spec, scalar_spec],
        out_shape=[jax.ShapeDtypeStruct((1, 1), jnp.float32)] * 3,
    )(*operands)

    loc_s = out[0][0, 0]
    nm = out[1][0, 0]
    ce = out[2][0, 0]
    total_loc = jnp.where(nm > 0.0, loc_s / jnp.maximum(nm, 1.0), loc_s)
    return total_loc + ce / float(_B)
